# Initial kernel scaffold; baseline (speedup 1.0000x reference)
import jax
import jax.numpy as jnp
from jax.experimental import pallas as pl


def kernel(x, edge_index, W1, W2):
    raise NotImplementedError("write your pallas kernel here")


# speedup vs baseline: 4.6798x; 4.6798x over previous
import functools

import jax
import jax.numpy as jnp
from jax import lax
from jax.experimental import pallas as pl
from jax.experimental.pallas import tpu as pltpu
from jax.experimental.pallas import tpu_sc as plsc

N = 10000
D = 128
E = 320000

_info = plsc.get_sparse_core_info()
NC = _info.num_cores
NS = _info.num_subcores
NW = NC * NS
EPW = E // NW
CH = 80
NCHUNK = EPW // CH
RPT = 624
ZR = 16

_mesh = plsc.VectorSubcoreMesh(core_axis_name="c", subcore_axis_name="s")


@functools.partial(
    pl.kernel,
    out_type=jax.ShapeDtypeStruct((NC, N, D), jnp.float32),
    mesh=_mesh,
    scratch_types=[
        pltpu.VMEM((CH,), jnp.int32),
        pltpu.VMEM((CH,), jnp.int32),
        pltpu.VMEM((CH, D), jnp.float32),
        pltpu.VMEM((ZR, D), jnp.float32),
        pltpu.VMEM_SHARED((N, D), jnp.float32),
        pltpu.SemaphoreType.DMA,
    ],
)
def _edge_agg(h_hbm, src_hbm, dst_hbm, out_hbm, src_v, dst_v, rows_v, zero_v,
              acc, sem):
    c = lax.axis_index("c")
    s = lax.axis_index("s")
    wid = s * NC + c

    zvec = jnp.zeros((16,), jnp.float32)
    for r in range(ZR):
        for k in range(D // 16):
            zero_v[r, pl.ds(k * 16, 16)] = zvec

    def zero_body(i, carry):
        pltpu.sync_copy(zero_v, acc.at[pl.ds(s * RPT + i * ZR, ZR)])
        return carry

    lax.fori_loop(0, RPT // ZR, zero_body, 0)

    @pl.when(s == NS - 1)
    def _():
        pltpu.sync_copy(zero_v, acc.at[pl.ds(NS * RPT, ZR)])

    plsc.subcore_barrier()

    base = wid * EPW

    def edge_body(j, carry):
        off = pl.multiple_of(base + j * CH, 8)
        pltpu.sync_copy(src_hbm.at[pl.ds(off, CH)], src_v)
        pltpu.sync_copy(dst_hbm.at[pl.ds(off, CH)], dst_v)
        pltpu.async_copy(h_hbm.at[src_v], rows_v, sem).wait()
        pltpu.sync_copy(rows_v, acc.at[dst_v], add=True)
        return carry

    lax.fori_loop(0, NCHUNK, edge_body, 0)
    plsc.subcore_barrier()

    pltpu.sync_copy(acc.at[pl.ds(s * RPT, RPT)],
                    out_hbm.at[c, pl.ds(s * RPT, RPT)])

    @pl.when(s == NS - 1)
    def _():
        pltpu.sync_copy(acc.at[pl.ds(NS * RPT, N - NS * RPT)],
                        out_hbm.at[c, pl.ds(NS * RPT, N - NS * RPT)])


_R = 1000


def _mm_kernel(x_ref, w_ref, o_ref):
    o_ref[...] = jnp.dot(x_ref[...], w_ref[...],
                         preferred_element_type=jnp.float32)


def _mm(x, W):
    return pl.pallas_call(
        _mm_kernel,
        grid=(N // _R,),
        in_specs=[pl.BlockSpec((_R, D), lambda i: (i, 0)),
                  pl.BlockSpec((D, D), lambda i: (0, 0))],
        out_specs=pl.BlockSpec((_R, D), lambda i: (i, 0)),
        out_shape=jax.ShapeDtypeStruct((N, D), jnp.float32),
    )(x, W)


def _relu_sum_mm_kernel(p_ref, w_ref, o_ref):
    h = jnp.maximum(p_ref[0] + p_ref[1], 0.0)
    o_ref[...] = jnp.dot(h, w_ref[...], preferred_element_type=jnp.float32)


def _relu_sum_mm(p, W):
    return pl.pallas_call(
        _relu_sum_mm_kernel,
        grid=(N // _R,),
        in_specs=[pl.BlockSpec((NC, _R, D), lambda i: (0, i, 0)),
                  pl.BlockSpec((D, D), lambda i: (0, 0))],
        out_specs=pl.BlockSpec((_R, D), lambda i: (i, 0)),
        out_shape=jax.ShapeDtypeStruct((N, D), jnp.float32),
    )(p, W)


def _relu_sum_kernel(p_ref, o_ref):
    o_ref[...] = jnp.maximum(p_ref[0] + p_ref[1], 0.0)


def _relu_sum(p):
    return pl.pallas_call(
        _relu_sum_kernel,
        grid=(N // _R,),
        in_specs=[pl.BlockSpec((NC, _R, D), lambda i: (0, i, 0))],
        out_specs=pl.BlockSpec((_R, D), lambda i: (i, 0)),
        out_shape=jax.ShapeDtypeStruct((N, D), jnp.float32),
    )(p)


def kernel(x, edge_index, W1, W2):
    src = edge_index[0]
    dst = edge_index[1]
    h1 = _mm(x, W1)
    p1 = _edge_agg(h1, src, dst)
    h2 = _relu_sum_mm(p1, W2)
    p2 = _edge_agg(h2, src, dst)
    return _relu_sum(p2)


# speedup vs baseline: 10.7097x; 2.2885x over previous
import functools

import jax
import jax.numpy as jnp
from jax import lax
from jax.experimental import pallas as pl
from jax.experimental.pallas import tpu as pltpu
from jax.experimental.pallas import tpu_sc as plsc

N = 10000
D = 128
E = 320000

_info = plsc.get_sparse_core_info()
NC = _info.num_cores
NS = _info.num_subcores
NW = NC * NS
CH = 125
NCH = E // NW // CH
GB = 16
NG = NCH // GB
RPT = 624
ZR = 16

_mesh = plsc.VectorSubcoreMesh(core_axis_name="c", subcore_axis_name="s")


@functools.partial(
    pl.kernel,
    out_type=jax.ShapeDtypeStruct((NC, N, D), jnp.float32),
    mesh=_mesh,
    scratch_types=[
        pltpu.VMEM((GB, CH), jnp.int32),
        pltpu.VMEM((GB, CH), jnp.int32),
        pltpu.VMEM((CH, D), jnp.float32),
        pltpu.VMEM((CH, D), jnp.float32),
        pltpu.VMEM((ZR, D), jnp.float32),
        pltpu.VMEM_SHARED((N, D), jnp.float32),
        pltpu.SemaphoreType.DMA,
        pltpu.SemaphoreType.DMA,
    ],
)
def _edge_agg(h_hbm, src_hbm, dst_hbm, out_hbm, src_b, dst_b, rows0, rows1,
              zero_v, acc, sem0, sem1):
    c = lax.axis_index("c")
    s = lax.axis_index("s")
    wid = s * NC + c

    zvec = jnp.zeros((16,), jnp.float32)
    for r in range(ZR):
        for k in range(D // 16):
            zero_v[r, pl.ds(k * 16, 16)] = zvec

    def zero_issue(i, carry):
        pltpu.async_copy(zero_v, acc.at[pl.ds(s * RPT + i * ZR, ZR)], sem0)
        return carry

    lax.fori_loop(0, RPT // ZR, zero_issue, 0)

    @pl.when(s == NS - 1)
    def _():
        pltpu.async_copy(zero_v, acc.at[pl.ds(NS * RPT, ZR)], sem0)

    def zero_drain(i, carry):
        pltpu.make_async_copy(zero_v, acc.at[pl.ds(0, ZR)], sem0).wait()
        return carry

    lax.fori_loop(0, RPT // ZR, zero_drain, 0)

    @pl.when(s == NS - 1)
    def _():
        pltpu.make_async_copy(zero_v, acc.at[pl.ds(0, ZR)], sem0).wait()

    plsc.subcore_barrier()

    row0 = pl.multiple_of(wid * NCH, 8)

    def _wait(rows, sem):
        pltpu.make_async_copy(h_hbm.at[src_b.at[0]], rows, sem).wait()

    def group_body(k, carry):
        grow = pl.multiple_of(row0 + k * GB, 8)
        pltpu.sync_copy(src_hbm.at[pl.ds(grow, GB)], src_b)
        pltpu.sync_copy(dst_hbm.at[pl.ds(grow, GB)], dst_b)
        pltpu.async_copy(h_hbm.at[src_b.at[0]], rows0, sem0)

        def edge_body(g, carry2):
            j = 2 * g
            pltpu.async_copy(h_hbm.at[src_b.at[j + 1]], rows1, sem1)
            _wait(rows0, sem0)
            pltpu.sync_copy(rows0, acc.at[dst_b.at[j]], add=True)

            @pl.when(g < GB // 2 - 1)
            def _():
                pltpu.async_copy(h_hbm.at[src_b.at[j + 2]], rows0, sem0)

            _wait(rows1, sem1)
            pltpu.sync_copy(rows1, acc.at[dst_b.at[j + 1]], add=True)
            return carry2

        lax.fori_loop(0, GB // 2, edge_body, 0)
        return carry

    lax.fori_loop(0, NG, group_body, 0)
    plsc.subcore_barrier()

    pltpu.sync_copy(acc.at[pl.ds(s * RPT, RPT)],
                    out_hbm.at[c, pl.ds(s * RPT, RPT)])

    @pl.when(s == NS - 1)
    def _():
        pltpu.sync_copy(acc.at[pl.ds(NS * RPT, N - NS * RPT)],
                        out_hbm.at[c, pl.ds(NS * RPT, N - NS * RPT)])


_R = 1000


def _mm_kernel(x_ref, w_ref, o_ref):
    o_ref[...] = jnp.dot(x_ref[...], w_ref[...],
                         preferred_element_type=jnp.float32)


def _mm(x, W):
    return pl.pallas_call(
        _mm_kernel,
        grid=(N // _R,),
        in_specs=[pl.BlockSpec((_R, D), lambda i: (i, 0)),
                  pl.BlockSpec((D, D), lambda i: (0, 0))],
        out_specs=pl.BlockSpec((_R, D), lambda i: (i, 0)),
        out_shape=jax.ShapeDtypeStruct((N, D), jnp.float32),
    )(x, W)


def _relu_sum_mm_kernel(p_ref, w_ref, o_ref):
    h = jnp.maximum(p_ref[0] + p_ref[1], 0.0)
    o_ref[...] = jnp.dot(h, w_ref[...], preferred_element_type=jnp.float32)


def _relu_sum_mm(p, W):
    return pl.pallas_call(
        _relu_sum_mm_kernel,
        grid=(N // _R,),
        in_specs=[pl.BlockSpec((NC, _R, D), lambda i: (0, i, 0)),
                  pl.BlockSpec((D, D), lambda i: (0, 0))],
        out_specs=pl.BlockSpec((_R, D), lambda i: (i, 0)),
        out_shape=jax.ShapeDtypeStruct((N, D), jnp.float32),
    )(p, W)


def _relu_sum_kernel(p_ref, o_ref):
    o_ref[...] = jnp.maximum(p_ref[0] + p_ref[1], 0.0)


def _relu_sum(p):
    return pl.pallas_call(
        _relu_sum_kernel,
        grid=(N // _R,),
        in_specs=[pl.BlockSpec((NC, _R, D), lambda i: (0, i, 0))],
        out_specs=pl.BlockSpec((_R, D), lambda i: (i, 0)),
        out_shape=jax.ShapeDtypeStruct((N, D), jnp.float32),
    )(p)


def kernel(x, edge_index, W1, W2):
    src = edge_index[0].reshape(NW * NCH, CH)
    dst = edge_index[1].reshape(NW * NCH, CH)
    h1 = _mm(x, W1)
    p1 = _edge_agg(h1, src, dst)
    h2 = _relu_sum_mm(p1, W2)
    p2 = _edge_agg(h2, src, dst)
    return _relu_sum(p2)
